# baseline (speedup 1.0000x reference)
import functools

import jax
import jax.numpy as jnp
from jax import lax
from jax.experimental import pallas as pl
from jax.experimental.pallas import tpu as pltpu
from jax.experimental.pallas import tpu_sc as plsc

F = 26
B = 4096
L = 20
V = 100001
D = 32

NC = 2
NS = 16
NW = NC * NS
CB = B // NW
CBL = CB * L


def _sc_kernel():
    mesh = plsc.VectorSubcoreMesh(core_axis_name="c", subcore_axis_name="s")

    @functools.partial(
        pl.kernel,
        mesh=mesh,
        out_type=jax.ShapeDtypeStruct((B, F, D), jnp.float32),
        compiler_params=pltpu.CompilerParams(
            needs_layout_passes=False, use_tc_tiling_on_sc=False),
        scratch_types=[
            pltpu.VMEM((CBL + 16,), jnp.int32),
            pltpu.VMEM((L, CB), jnp.int32),
            pltpu.VMEM((CBL, D), jnp.float32),
            pltpu.VMEM((CB, D), jnp.float32),
            pltpu.SemaphoreType.DMA,
        ],
    )
    def k(x_hbm, t_hbm, out_hbm, raw_v, adj_v, rows_v, outb_v, sem):
        wid = lax.axis_index("s") * NC + lax.axis_index("c")
        b0 = wid * CB
        lanes = lax.iota(jnp.int32, 16)

        @pl.loop(0, F)
        def per_field(f):
            start = f * (B * L) + wid * CBL
            pltpu.sync_copy(x_hbm.at[pl.ds(start, CBL)], raw_v.at[pl.ds(0, CBL)])

            fv = f * V
            for r in range(L):
                for c in range(CB // 16):
                    v = raw_v[pl.ds(r * CB + c * 16, 16)]
                    adj_v[r, pl.ds(c * 16, 16)] = v + fv

            copies = []
            for r in range(L):
                copies.append(
                    pltpu.async_copy(
                        t_hbm.at[adj_v.at[r]],
                        rows_v.at[pl.ds(r * CB, CB)],
                        sem,
                    )
                )
            for cp in copies:
                cp.wait()

            tail_mask = lanes < (L - 16)

            @pl.loop(0, CB // 16)
            def per_bag_group(jb):
                for t in range(16):
                    j = jb * 16 + t
                    q0 = j * L
                    iv0 = raw_v[pl.ds(q0, 16)]
                    iv1 = raw_v[pl.ds(q0 + 16, 16)]
                    nz = (jnp.where(iv0 != 0, 1, 0)
                          + jnp.where(jnp.logical_and(iv1 != 0, tail_mask), 1, 0))
                    cntf = plsc.cumsum(nz).astype(jnp.float32)
                    rv = 1.0 / jnp.maximum(cntf, 1.0)
                    r = rv[15]

                    acc0 = rows_v[q0, pl.ds(0, 16)]
                    acc1 = rows_v[q0, pl.ds(16, 16)]
                    for l in range(1, L):
                        acc0 = acc0 + rows_v[q0 + l, pl.ds(0, 16)]
                        acc1 = acc1 + rows_v[q0 + l, pl.ds(16, 16)]
                    outb_v[j, pl.ds(0, 16)] = acc0 * r
                    outb_v[j, pl.ds(16, 16)] = acc1 * r

            pltpu.sync_copy(outb_v, out_hbm.at[pl.ds(b0, CB), f])

    return k


def kernel(x, tables):
    xf = x.reshape(F * B * L)
    tf = tables.reshape(F * V, D)
    return _sc_kernel()(xf, tf)


# speedup vs baseline: 2.4363x; 2.4363x over previous
import functools

import jax
import jax.numpy as jnp
from jax import lax
from jax.experimental import pallas as pl
from jax.experimental.pallas import tpu as pltpu
from jax.experimental.pallas import tpu_sc as plsc

F = 26
B = 4096
L = 20
V = 100001
D = 32

NC = 2
NS = 16
NW = NC * NS
CB = B // NW
CBL = CB * L


def _sc_kernel():
    mesh = plsc.VectorSubcoreMesh(core_axis_name="c", subcore_axis_name="s")

    @functools.partial(
        pl.kernel,
        mesh=mesh,
        out_type=jax.ShapeDtypeStruct((B, F, D), jnp.float32),
        compiler_params=pltpu.CompilerParams(
            needs_layout_passes=False, use_tc_tiling_on_sc=False),
        scratch_types=[
            pltpu.VMEM((CBL + 16,), jnp.int32),
            pltpu.VMEM((L, CB), jnp.int32),
            pltpu.VMEM((CBL, D), jnp.float32),
            pltpu.VMEM((CB, D), jnp.float32),
            pltpu.SemaphoreType.DMA,
        ],
    )
    def k(x_hbm, t_hbm, out_hbm, raw_v, idx_v, rows_v, outb_v, sem):
        wid = lax.axis_index("s") * NC + lax.axis_index("c")
        b0 = wid * CB
        lanes = lax.iota(jnp.int32, 16)
        tail_mask = lanes < (L - 16)

        @pl.loop(0, F)
        def per_field(f):
            start = f * (B * L) + wid * CBL
            pltpu.sync_copy(x_hbm.at[pl.ds(start, CBL)], raw_v.at[pl.ds(0, CBL)])

            for r in range(L):
                for c in range(CB // 16):
                    idx_v[r, pl.ds(c * 16, 16)] = raw_v[pl.ds(r * CB + c * 16, 16)]

            copies = []
            for r in range(L):
                copies.append(
                    pltpu.async_copy(
                        t_hbm.at[f].at[idx_v.at[r]],
                        rows_v.at[pl.ds(r * CB, CB)],
                        sem,
                    )
                )
            for cp in copies:
                cp.wait()

            @pl.loop(0, CB // 16)
            def per_bag_group(jb):
                for t in range(16):
                    j = jb * 16 + t
                    q0 = j * L
                    iv0 = raw_v[pl.ds(q0, 16)]
                    iv1 = raw_v[pl.ds(q0 + 16, 16)]
                    nz = (jnp.where(iv0 != 0, 1, 0)
                          + jnp.where(jnp.logical_and(iv1 != 0, tail_mask), 1, 0))
                    cntf = plsc.cumsum(nz).astype(jnp.float32)
                    rv = 1.0 / jnp.maximum(cntf, 1.0)
                    r = rv[15]

                    acc0 = rows_v[q0, pl.ds(0, 16)]
                    acc1 = rows_v[q0, pl.ds(16, 16)]
                    for l in range(1, L):
                        acc0 = acc0 + rows_v[q0 + l, pl.ds(0, 16)]
                        acc1 = acc1 + rows_v[q0 + l, pl.ds(16, 16)]
                    outb_v[j, pl.ds(0, 16)] = acc0 * r
                    outb_v[j, pl.ds(16, 16)] = acc1 * r

            pltpu.sync_copy(outb_v, out_hbm.at[pl.ds(b0, CB), f])

    return k


def kernel(x, tables):
    xf = x.reshape(F * B * L)
    return _sc_kernel()(xf, tables)


# speedup vs baseline: 5.6808x; 2.3317x over previous
import functools

import jax
import jax.numpy as jnp
from jax import lax
from jax.experimental import pallas as pl
from jax.experimental.pallas import tpu as pltpu
from jax.experimental.pallas import tpu_sc as plsc

F = 26
B = 4096
L = 20
V = 100001
D = 32

NC = 2
NS = 16
NW = NC * NS
CB = B // NW
CBL = CB * L


def _sc_kernel():
    mesh = plsc.VectorSubcoreMesh(core_axis_name="c", subcore_axis_name="s")

    @functools.partial(
        pl.kernel,
        mesh=mesh,
        out_type=jax.ShapeDtypeStruct((B, F, D), jnp.float32),
        compiler_params=pltpu.CompilerParams(
            needs_layout_passes=False, use_tc_tiling_on_sc=False),
        scratch_types=[
            pltpu.VMEM((CBL + 16,), jnp.int32),
            pltpu.VMEM((CBL, D), jnp.float32),
            pltpu.VMEM((CB, D), jnp.float32),
            pltpu.SemaphoreType.DMA,
        ],
    )
    def k(x_hbm, *refs):
        t_hbm = refs[:F]
        out_hbm = refs[F]
        raw_v, rows_v, outb_v, sem = refs[F + 1:]
        wid = lax.axis_index("s") * NC + lax.axis_index("c")
        b0 = wid * CB
        lanes = lax.iota(jnp.int32, 16)
        tail_mask = lanes < (L - 16)

        def per_bag(j):
            q0 = j * L
            iv0 = raw_v[pl.ds(q0, 16)]
            iv1 = raw_v[pl.ds(q0 + 16, 16)]
            nz = (jnp.where(iv0 != 0, 1, 0)
                  + jnp.where(jnp.logical_and(iv1 != 0, tail_mask), 1, 0))
            cntf = plsc.cumsum(nz).astype(jnp.float32)
            rv = 1.0 / jnp.maximum(cntf, 1.0)
            r = rv[15]

            acc0 = rows_v[q0, pl.ds(0, 16)]
            acc1 = rows_v[q0, pl.ds(16, 16)]
            for l in range(1, L):
                acc0 = acc0 + rows_v[q0 + l, pl.ds(0, 16)]
                acc1 = acc1 + rows_v[q0 + l, pl.ds(16, 16)]
            outb_v[j, pl.ds(0, 16)] = acc0 * r
            outb_v[j, pl.ds(16, 16)] = acc1 * r

        for f in range(F):
            start = f * (B * L) + wid * CBL
            pltpu.sync_copy(x_hbm.at[pl.ds(start, CBL)],
                            raw_v.at[pl.ds(0, CBL)])

            copies = []
            for r in range(L):
                copies.append(
                    pltpu.async_copy(
                        t_hbm[f].at[raw_v.at[pl.ds(r * CB, CB)]],
                        rows_v.at[pl.ds(r * CB, CB)],
                        sem,
                    )
                )
            for cp in copies:
                cp.wait()

            pl.loop(0, CB)(per_bag)

            pltpu.sync_copy(outb_v, out_hbm.at[pl.ds(b0, CB), f])

    return k


def kernel(x, tables):
    xf = x.reshape(F * B * L)
    return _sc_kernel()(xf, *[tables[f] for f in range(F)])
